# baseline (speedup 1.0000x reference)
import functools

import jax
import jax.numpy as jnp
from jax import lax
from jax.experimental import pallas as pl
from jax.experimental.pallas import tpu as pltpu
from jax.experimental.pallas import tpu_sc as plsc

B = 16384
D = 1000
NC = 2
NS = 16
NW = NC * NS
ROWS_PER_W = B // NW
CHUNK = 64
NCHUNK = ROWS_PER_W // CHUNK
GROUPS = CHUNK // 16

_mesh = plsc.VectorSubcoreMesh(core_axis_name="c", subcore_axis_name="s")


@functools.partial(
    pl.kernel,
    out_type=jax.ShapeDtypeStruct((B * D,), jnp.float32),
    mesh=_mesh,
    scratch_types=[
        pltpu.VMEM((ROWS_PER_W,), jnp.int32),
        pltpu.VMEM((CHUNK * D,), jnp.float32),
    ],
    compiler_params=pltpu.CompilerParams(needs_layout_passes=False),
)
def _onehot_sc(x_hbm, out_hbm, idx_v, buf):
    wid = lax.axis_index("s") * NC + lax.axis_index("c")
    base = wid * ROWS_PER_W
    pltpu.sync_copy(x_hbm.at[pl.ds(base, ROWS_PER_W)], idx_v)

    z16 = jnp.zeros((16,), jnp.float32)
    ones16 = jnp.ones((16,), jnp.float32)
    iota16 = lax.iota(jnp.int32, 16)

    def _zero_step(i, carry):
        buf[pl.ds(i * 16, 16)] = z16
        return carry

    lax.fori_loop(0, (CHUNK * D) // 16, _zero_step, 0)

    for c in range(NCHUNK):
        flats = []
        for g in range(GROUPS):
            col = idx_v[pl.ds(c * CHUNK + g * 16, 16)]
            flat = (iota16 + g * 16) * D + col
            flats.append(flat)
            plsc.store_scatter(buf, [flat], ones16)
        pltpu.sync_copy(
            buf, out_hbm.at[pl.ds((base + c * CHUNK) * D, CHUNK * D)]
        )
        for flat in flats:
            plsc.store_scatter(buf, [flat], z16)


def kernel(x):
    out_flat = _onehot_sc(x.astype(jnp.int32))
    return out_flat.reshape(B, D)


# speedup vs baseline: 1.7677x; 1.7677x over previous
import functools

import jax
import jax.numpy as jnp
from jax import lax
from jax.experimental import pallas as pl
from jax.experimental.pallas import tpu as pltpu
from jax.experimental.pallas import tpu_sc as plsc

B = 16384
D = 1000
NC = 2
NS = 16
NW = NC * NS
ROWS_PER_W = B // NW
CHUNK = 64
NCHUNK = ROWS_PER_W // CHUNK
GROUPS = CHUNK // 16

_mesh = plsc.VectorSubcoreMesh(core_axis_name="c", subcore_axis_name="s")


@functools.partial(
    pl.kernel,
    out_type=jax.ShapeDtypeStruct((B, D), jnp.float32),
    mesh=_mesh,
    scratch_types=[
        pltpu.VMEM((ROWS_PER_W,), jnp.int32),
        pltpu.VMEM((CHUNK, D), jnp.float32),
    ],
    compiler_params=pltpu.CompilerParams(needs_layout_passes=False),
)
def _onehot_sc(x_hbm, out_hbm, idx_v, buf):
    wid = lax.axis_index("s") * NC + lax.axis_index("c")
    base = wid * ROWS_PER_W
    pltpu.sync_copy(x_hbm.at[pl.ds(base, ROWS_PER_W)], idx_v)

    z16 = jnp.zeros((16,), jnp.float32)
    ones16 = jnp.ones((16,), jnp.float32)
    iota16 = lax.iota(jnp.int32, 16)

    def _zero_row(r, carry):
        for j in range(D // 16):
            buf[r, pl.ds(j * 16, 16)] = z16
        buf[r, pl.ds(D - 16, 16)] = z16
        return carry

    lax.fori_loop(0, CHUNK, _zero_row, 0)

    for c in range(NCHUNK):
        pairs = []
        for g in range(GROUPS):
            col = idx_v[pl.ds(c * CHUNK + g * 16, 16)]
            row = iota16 + g * 16
            pairs.append((row, col))
            plsc.store_scatter(buf, [row, col], ones16)
        pltpu.sync_copy(buf, out_hbm.at[pl.ds(base + c * CHUNK, CHUNK)])
        for row, col in pairs:
            plsc.store_scatter(buf, [row, col], z16)


def kernel(x):
    return _onehot_sc(x.astype(jnp.int32))


# speedup vs baseline: 3.8132x; 2.1571x over previous
import functools

import jax
import jax.numpy as jnp
from jax import lax
from jax.experimental import pallas as pl
from jax.experimental.pallas import tpu as pltpu
from jax.experimental.pallas import tpu_sc as plsc

B = 16384
D = 1000
NC = 2
NS = 16
NW = NC * NS
COLS_PER_W = B // NW
RCHUNK = 128
CHUNKS = [(c * RCHUNK, min(RCHUNK, D - c * RCHUNK)) for c in range(8)]
GROUPS = COLS_PER_W // 16

_mesh = plsc.VectorSubcoreMesh(core_axis_name="c", subcore_axis_name="s")


@functools.partial(
    pl.kernel,
    out_type=jax.ShapeDtypeStruct((D, B), jnp.float32),
    mesh=_mesh,
    scratch_types=[
        pltpu.VMEM((COLS_PER_W,), jnp.int32),
        pltpu.VMEM((RCHUNK, COLS_PER_W), jnp.float32),
    ],
    compiler_params=pltpu.CompilerParams(needs_layout_passes=False),
)
def _onehot_sc(x_hbm, out_hbm, idx_v, buf):
    wid = lax.axis_index("s") * NC + lax.axis_index("c")
    col_base = wid * COLS_PER_W
    pltpu.sync_copy(x_hbm.at[pl.ds(col_base, COLS_PER_W)], idx_v)

    z16 = jnp.zeros((16,), jnp.float32)
    ones16 = jnp.ones((16,), jnp.float32)
    iota16 = lax.iota(jnp.int32, 16)

    def _zero_row(r, carry):
        for j in range(COLS_PER_W // 16):
            buf[r, pl.ds(j * 16, 16)] = z16
        return carry

    lax.fori_loop(0, RCHUNK, _zero_row, 0)

    for r0, size in CHUNKS:

        def _scatter(g, val):
            x16 = idx_v[pl.ds(g * 16, 16)]
            rows = x16 - r0
            cols = iota16 + g * 16
            mask = (x16 >= r0) & (x16 < r0 + size)
            plsc.store_scatter(buf, [rows, cols], val, mask=mask)
            return val

        lax.fori_loop(0, GROUPS, _scatter, ones16)
        pltpu.sync_copy(
            buf.at[pl.ds(0, size)],
            out_hbm.at[pl.ds(r0, size), pl.ds(col_base, COLS_PER_W)],
        )
        lax.fori_loop(0, GROUPS, _scatter, z16)


def kernel(x):
    return _onehot_sc(x.astype(jnp.int32)).T


# speedup vs baseline: 4.0967x; 1.0744x over previous
import functools

import jax
import jax.numpy as jnp
from jax import lax
from jax.experimental import pallas as pl
from jax.experimental.pallas import tpu as pltpu
from jax.experimental.pallas import tpu_sc as plsc

B = 16384
D = 1000
NC = 2
NS = 16
NW = NC * NS
COLS_PER_W = B // NW
RCHUNK = 120
CHUNKS = [(c * RCHUNK, min(RCHUNK, D - c * RCHUNK))
          for c in range((D + RCHUNK - 1) // RCHUNK)]
GROUPS = COLS_PER_W // 16
UNROLL = 4

_mesh = plsc.VectorSubcoreMesh(core_axis_name="c", subcore_axis_name="s")


@functools.partial(
    pl.kernel,
    out_type=jax.ShapeDtypeStruct((D, B), jnp.float32),
    mesh=_mesh,
    scratch_types=[
        pltpu.VMEM((COLS_PER_W,), jnp.int32),
        pltpu.VMEM((RCHUNK, COLS_PER_W), jnp.float32),
        pltpu.VMEM((RCHUNK, COLS_PER_W), jnp.float32),
        pltpu.SemaphoreType.DMA,
        pltpu.SemaphoreType.DMA,
    ],
    compiler_params=pltpu.CompilerParams(needs_layout_passes=False),
)
def _onehot_sc(x_hbm, out_hbm, idx_v, buf0, buf1, sem0, sem1):
    wid = lax.axis_index("s") * NC + lax.axis_index("c")
    col_base = wid * COLS_PER_W
    pltpu.sync_copy(x_hbm.at[pl.ds(col_base, COLS_PER_W)], idx_v)

    bufs = (buf0, buf1)
    sems = (sem0, sem1)
    z16 = jnp.zeros((16,), jnp.float32)
    ones16 = jnp.ones((16,), jnp.float32)
    iota16 = lax.iota(jnp.int32, 16)

    def _zero(buf):
        def _zero_row(r, carry):
            for j in range(COLS_PER_W // 16):
                buf[r, pl.ds(j * 16, 16)] = z16
            return carry

        lax.fori_loop(0, RCHUNK, _zero_row, 0)

    def _scatter(buf, r0, size, val):
        usize = jnp.full((16,), size, jnp.uint32)

        def _step(i, carry):
            for k in range(UNROLL):
                off = (i * UNROLL + k) * 16
                x16 = idx_v[pl.ds(off, 16)]
                rows = x16 - r0
                mask = plsc.bitcast(rows, jnp.uint32) < usize
                cols = iota16 + off
                plsc.store_scatter(buf, [rows, cols], val, mask=mask)
            return carry

        lax.fori_loop(0, GROUPS // UNROLL, _step, 0)

    def _start(b, r0, size):
        return pltpu.async_copy(
            bufs[b].at[pl.ds(0, size)],
            out_hbm.at[pl.ds(r0, size), pl.ds(col_base, COLS_PER_W)],
            sems[b],
        )

    _zero(bufs[0])
    r0, size = CHUNKS[0]
    _scatter(bufs[0], r0, size, ones16)
    handles = [_start(0, r0, size), None]
    pending = [(r0, size), None]
    _zero(bufs[1])

    for i in range(1, len(CHUNKS)):
        b = i % 2
        if pending[b] is not None:
            handles[b].wait()
            pr0, psize = pending[b]
            _scatter(bufs[b], pr0, psize, z16)
        r0, size = CHUNKS[i]
        _scatter(bufs[b], r0, size, ones16)
        handles[b] = _start(b, r0, size)
        pending[b] = (r0, size)

    handles[0].wait()
    handles[1].wait()


def kernel(x):
    return _onehot_sc(x.astype(jnp.int32)).T
